# baseline (speedup 1.0000x reference)
import functools

import jax
import jax.numpy as jnp
import numpy as np
from jax import lax
from jax.experimental import pallas as pl
from jax.experimental.pallas import tpu as pltpu
from jax.experimental.pallas import tpu_sc as plsc

_EMB = 32
_LEN = 512
_LB = -15.0
_UB = 16.0
_LN_BASE = float(np.log(256.0 ** (1.0 / 15.0)))

_DD = 640
_RLEN = 128 * _DD


def _bucket(d):
    sign = jnp.sign(d).astype(jnp.float32)
    a = jnp.abs(d).astype(jnp.float32)
    v = jnp.floor(jnp.log(a) / _LN_BASE + 1.0)
    v = jnp.where(v < 0, 0.0, v)
    v = v * sign
    v = jnp.where(v < _LB, _LB, v)
    v = jnp.where(v > _UB, _UB, v)
    return (v - _LB).astype(jnp.int32)


def _m3_body(o_ref):
    sh = (8, 80, 8, 128)
    rr = lax.broadcasted_iota(jnp.int32, sh, 0)
    flat = (
        lax.broadcasted_iota(jnp.int32, sh, 1) * 1024
        + lax.broadcasted_iota(jnp.int32, sh, 2) * 128
        + lax.broadcasted_iota(jnp.int32, sh, 3)
    )
    kt = (flat // (8 * _DD)) % 4
    e = flat // (32 * _DD) * 8 + (flat // _DD) % 8
    dd = flat % _DD
    s_r = (_LEN - 1 - rr) & 7
    d = 128 * kt + dd + s_r - (_LEN - 1)
    o_ref[...] = (_bucket(d) == e).astype(jnp.float32)


_build_m3 = pl.pallas_call(
    _m3_body,
    out_shape=jax.ShapeDtypeStruct((8, 80, 8, 128), jnp.float32),
)

_NUM_CORES = 2
_PPW = 64


def _fanout_body(m3_hbm, out_hbm, m3_v, sem):
    wid = lax.axis_index("s") * _NUM_CORES + lax.axis_index("c")
    r = lax.rem(wid, 8)
    b = wid // 8
    s_r = lax.rem(_LEN - 1 - r, 8)

    def load(i, carry):
        pltpu.async_copy(m3_hbm.at[pl.ds(r * _RLEN + i * _DD, _DD)], m3_v.at[i], sem)
        return carry

    lax.fori_loop(0, 128, load, 0)

    def drain_load(i, carry):
        pltpu.make_async_copy(m3_hbm.at[pl.ds(0, _DD)], m3_v.at[0], sem).wait()
        return carry

    lax.fori_loop(0, 128, drain_load, 0)

    def fire(m, carry):
        j = r + 8 * m
        t = b * _LEN + j
        q = pl.multiple_of(_LEN - 1 - j - s_r, 8)
        pltpu.async_copy(m3_v.at[:, pl.ds(q, 128)], out_hbm.at[t], sem)
        return carry

    lax.fori_loop(0, _PPW, fire, 0)

    def drain(i, carry):
        pltpu.make_async_copy(m3_v.at[:, pl.ds(0, 128)], out_hbm.at[0], sem).wait()
        return carry

    lax.fori_loop(0, _PPW, drain, 0)


@functools.cache
def _get_fanout():
    return functools.partial(
        pl.kernel,
        out_type=jax.ShapeDtypeStruct((4 * _LEN, 128, 128), jnp.float32),
        mesh=plsc.VectorSubcoreMesh(
            core_axis_name="c",
            subcore_axis_name="s",
            num_cores=_NUM_CORES,
            num_subcores=16,
        ),
        scratch_types=[pltpu.VMEM((128, _DD), jnp.float32), pltpu.SemaphoreType.DMA],
        compiler_params=pltpu.CompilerParams(use_tc_tiling_on_sc=False),
    )(_fanout_body)


def kernel(x):
    batch, length = x.shape
    m3 = _build_m3().reshape(8 * _RLEN)
    out6 = _get_fanout()(m3)
    out6 = out6.reshape(batch, length, 4, 4, 8, 128)
    return out6.transpose(0, 1, 3, 5, 2, 4).reshape(batch, length, length, _EMB)


# speedup vs baseline: 1.6535x; 1.6535x over previous
import functools

import jax
import jax.numpy as jnp
import numpy as np
from jax import lax
from jax.experimental import pallas as pl
from jax.experimental.pallas import tpu as pltpu

_EMB = 32
_LEN = 512
_LB = -15.0
_UB = 16.0
_LN_BASE = float(np.log(256.0 ** (1.0 / 15.0)))

_BJ = 16


def _bucket(d):
    sign = jnp.sign(d).astype(jnp.float32)
    a = jnp.abs(d).astype(jnp.float32)
    v = jnp.floor(jnp.log(a) / _LN_BASE + 1.0)
    v = jnp.where(v < 0, 0.0, v)
    v = v * sign
    v = jnp.where(v < _LB, _LB, v)
    v = jnp.where(v > _UB, _UB, v)
    return (v - _LB).astype(jnp.int32)


def _plane_body(o_ref, lohi_ref):
    jb = pl.program_id(0)

    @pl.when(jb == 0)
    def _():
        dd = lax.broadcasted_iota(jnp.int32, (_EMB, 1024), 1) - (_LEN - 1)
        e = lax.broadcasted_iota(jnp.int32, (_EMB, 1024), 0)
        m = _bucket(dd) == e
        dfl = dd.astype(jnp.float32)
        lohi_ref[:, 0:1] = jnp.min(jnp.where(m, dfl, 1e9), axis=1, keepdims=True)
        lohi_ref[:, 1:2] = jnp.max(jnp.where(m, dfl, -1e9), axis=1, keepdims=True)

    lo = lohi_ref[:, 0:1].reshape(1, 1, _EMB, 1)
    hi = lohi_ref[:, 1:2].reshape(1, 1, _EMB, 1)
    kk = lax.broadcasted_iota(jnp.int32, (1, _BJ, _EMB, _LEN), 3)
    jj = lax.broadcasted_iota(jnp.int32, (1, _BJ, _EMB, _LEN), 1)
    d = (kk - jj - jb * _BJ).astype(jnp.float32)
    v = jnp.clip(jnp.minimum(d - lo + 1.0, hi - d + 1.0), 0.0, 1.0)
    o_ref[...] = jnp.broadcast_to(v, (4, _BJ, _EMB, _LEN))


_planes = pl.pallas_call(
    _plane_body,
    grid=(_LEN // _BJ,),
    out_specs=pl.BlockSpec((4, _BJ, _EMB, _LEN), lambda j: (0, j, 0, 0)),
    out_shape=jax.ShapeDtypeStruct((4, _LEN, _EMB, _LEN), jnp.float32),
    scratch_shapes=[pltpu.VMEM((_EMB, 128), jnp.float32)],
)


def kernel(x):
    batch, length = x.shape
    out = _planes()
    return jnp.transpose(out, (0, 1, 3, 2))


# speedup vs baseline: 1.6582x; 1.0028x over previous
import jax
import jax.numpy as jnp
import numpy as np
from jax import lax
from jax.experimental import pallas as pl
from jax.experimental.pallas import tpu as pltpu

_EMB = 32
_LEN = 512
_LB = -15.0
_UB = 16.0
_LN_BASE = float(np.log(256.0 ** (1.0 / 15.0)))

_BJ = 16


def _bucket(d):
    sign = jnp.sign(d).astype(jnp.float32)
    a = jnp.abs(d).astype(jnp.float32)
    v = jnp.floor(jnp.log(a) / _LN_BASE + 1.0)
    v = jnp.where(v < 0, 0.0, v)
    v = v * sign
    v = jnp.where(v < _LB, _LB, v)
    v = jnp.where(v > _UB, _UB, v)
    return (v - _LB).astype(jnp.int32)


def _plane_body(o_ref, lohi_ref):
    jb = pl.program_id(0)

    @pl.when(jb == 0)
    def _():
        dd = lax.broadcasted_iota(jnp.int32, (_EMB, 1024), 1) - (_LEN - 1)
        e = lax.broadcasted_iota(jnp.int32, (_EMB, 1024), 0)
        m = _bucket(dd) == e
        dfl = dd.astype(jnp.float32)
        lohi_ref[:, 0:1] = jnp.min(jnp.where(m, dfl, 1e9), axis=1, keepdims=True)
        lohi_ref[:, 1:2] = jnp.max(jnp.where(m, dfl, -1e9), axis=1, keepdims=True)

    lo = lohi_ref[:, 0:1].reshape(1, 1, _EMB, 1)
    hi = lohi_ref[:, 1:2].reshape(1, 1, _EMB, 1)
    kk = lax.broadcasted_iota(jnp.int32, (1, _BJ, _EMB, _LEN), 3)
    jj = lax.broadcasted_iota(jnp.int32, (1, _BJ, _EMB, _LEN), 1)
    d = (kk - jj - jb * _BJ).astype(jnp.float32)
    v = jnp.clip(jnp.minimum(d - lo + 1.0, hi - d + 1.0), 0.0, 1.0)
    o_ref[...] = jnp.broadcast_to(v, (4, _BJ, _EMB, _LEN))


_planes = pl.pallas_call(
    _plane_body,
    grid=(_LEN // _BJ,),
    out_specs=pl.BlockSpec((4, _BJ, _EMB, _LEN), lambda j: (0, j, 0, 0)),
    out_shape=jax.ShapeDtypeStruct((4, _LEN, _EMB, _LEN), jnp.float32),
    scratch_shapes=[pltpu.VMEM((_EMB, 128), jnp.float32)],
)


def kernel(x):
    batch, length = x.shape
    out = _planes()
    return jnp.transpose(out, (0, 1, 3, 2))
